# baseline (speedup 1.0000x reference)
import functools

import jax
import jax.numpy as jnp
from jax import lax
from jax.experimental import pallas as pl
from jax.experimental.pallas import tpu as pltpu
from jax.experimental.pallas import tpu_sc as plsc

N = 10000
E = 320000
D = 128
NC = 2
NS = 16
NW = NC * NS
EPW = E // NW
CH = 40
NCH = EPW // CH
RPT = 624
RPT_LAST = N - 15 * RPT


NB = 9
LI = 7
LG = 6


def _sc_agg_body(x_hbm, e_hbm, z_hbm, out0_hbm, out1_hbm, *scr):
    src_c = scr[0:NB]
    dst_c = scr[NB:2 * NB]
    rows = scr[2 * NB:3 * NB]
    acc = scr[3 * NB]
    isems = scr[3 * NB + 1:3 * NB + 1 + NB]
    gsems = scr[3 * NB + 1 + NB:3 * NB + 1 + 2 * NB]
    ssems = scr[3 * NB + 1 + 2 * NB:3 * NB + 1 + 3 * NB]
    c = lax.axis_index("c")
    s = lax.axis_index("s")
    wid = s * NC + c
    ebase = wid * EPW

    zsem = scr[3 * NB + 1 + 3 * NB]

    @pl.when(s < 15)
    def _():
        pltpu.async_copy(z_hbm.at[pl.ds(0, RPT)],
                         acc.at[pl.ds(s * RPT, RPT)], zsem)

    @pl.when(s == 15)
    def _():
        pltpu.async_copy(z_hbm, acc.at[pl.ds(15 * RPT, RPT_LAST)], zsem)

    def fire_idx(b, k):
        pltpu.async_copy(e_hbm.at[pl.ds(ebase + k * CH, CH)],
                         src_c[b], isems[b])
        pltpu.async_copy(e_hbm.at[pl.ds(E + ebase + k * CH, CH)],
                         dst_c[b], isems[b])

    def wait_idx(b, k):
        pltpu.make_async_copy(e_hbm.at[pl.ds(ebase + k * CH, CH)],
                              src_c[b], isems[b]).wait()
        pltpu.make_async_copy(e_hbm.at[pl.ds(E + ebase + k * CH, CH)],
                              dst_c[b], isems[b]).wait()

    def fire_gather(b):
        pltpu.async_copy(x_hbm.at[src_c[b]], rows[b], gsems[b])

    def wait_gather(b):
        pltpu.make_async_copy(x_hbm.at[src_c[b]], rows[b], gsems[b]).wait()

    def fire_scatter(b):
        pltpu.async_copy(rows[b], acc.at[dst_c[b]], ssems[b], add=True)

    def wait_scatter(b):
        pltpu.make_async_copy(rows[b], acc.at[dst_c[b]], ssems[b]).wait()

    for k in range(LI):
        fire_idx(k % NB, k)
    for k in range(LG):
        wait_idx(k % NB, k)
        fire_gather(k % NB)

    @pl.when(s < 15)
    def _():
        pltpu.make_async_copy(z_hbm.at[pl.ds(0, RPT)],
                              acc.at[pl.ds(s * RPT, RPT)], zsem).wait()

    @pl.when(s == 15)
    def _():
        pltpu.make_async_copy(z_hbm, acc.at[pl.ds(15 * RPT, RPT_LAST)],
                              zsem).wait()

    plsc.subcore_barrier()

    @pl.loop(0, ((NCH + NB - 1) // NB) * NB, step=NB)
    def _grp(i):
        for b in range(NB):
            j = i + b

            @pl.when(j < NCH)
            def _():
                @pl.when(j >= NB - LI)
                def _():
                    wait_scatter((b + LI) % NB)

                @pl.when(j + LI < NCH)
                def _():
                    fire_idx((b + LI) % NB, j + LI)

                @pl.when(j + LG < NCH)
                def _():
                    wait_idx((b + LG) % NB, j + LG)
                    fire_gather((b + LG) % NB)

                wait_gather(b)
                fire_scatter(b)

    for jj in range(NCH - (NB - LI), NCH):
        wait_scatter(jj % NB)

    plsc.subcore_barrier()

    @pl.when(jnp.logical_and(s < 15, c == 0))
    def _():
        pltpu.sync_copy(acc.at[pl.ds(s * RPT, RPT)],
                        out0_hbm.at[pl.ds(s * RPT, RPT)])

    @pl.when(jnp.logical_and(s == 15, c == 0))
    def _():
        pltpu.sync_copy(acc.at[pl.ds(15 * RPT, RPT_LAST)],
                        out0_hbm.at[pl.ds(15 * RPT, RPT_LAST)])

    @pl.when(jnp.logical_and(s < 15, c == 1))
    def _():
        pltpu.sync_copy(acc.at[pl.ds(s * RPT, RPT)],
                        out1_hbm.at[pl.ds(s * RPT, RPT)])

    @pl.when(jnp.logical_and(s == 15, c == 1))
    def _():
        pltpu.sync_copy(acc.at[pl.ds(15 * RPT, RPT_LAST)],
                        out1_hbm.at[pl.ds(15 * RPT, RPT_LAST)])


_sc_agg = functools.partial(
    pl.kernel,
    out_type=(jax.ShapeDtypeStruct((N, D), jnp.float32),
              jax.ShapeDtypeStruct((N, D), jnp.float32)),
    mesh=plsc.VectorSubcoreMesh(core_axis_name="c", subcore_axis_name="s",
                                num_cores=NC, num_subcores=NS),
    scratch_types=(
        [pltpu.VMEM((CH,), jnp.int32)] * (2 * NB)
        + [pltpu.VMEM((CH, D), jnp.float32)] * NB
        + [pltpu.VMEM_SHARED((N, D), jnp.float32)]
        + [pltpu.SemaphoreType.DMA] * (3 * NB + 1)
    ),
)(_sc_agg_body)


def _mlp_body(scale_ref, x_ref, p0_ref, p1_ref, w1_ref, b1_ref, w2_ref,
              b2_ref, o_ref):
    z = scale_ref[0, 0] * x_ref[...] + p0_ref[...] + p1_ref[...]
    h = lax.dot_general(z, w1_ref[...], (((1,), (1,)), ((), ())),
                        preferred_element_type=jnp.float32)
    h = jnp.maximum(h + b1_ref[...], 0.0)
    o = lax.dot_general(h, w2_ref[...], (((1,), (1,)), ((), ())),
                        preferred_element_type=jnp.float32)
    o_ref[...] = o + b2_ref[...]


BM = 2000

_mlp = pl.pallas_call(
    _mlp_body,
    grid=(N // BM,),
    compiler_params=pltpu.CompilerParams(
        dimension_semantics=("parallel",)),
    in_specs=[
        pl.BlockSpec(memory_space=pltpu.SMEM),
        pl.BlockSpec((BM, D), lambda i: (i, 0)),
        pl.BlockSpec((BM, D), lambda i: (i, 0)),
        pl.BlockSpec((BM, D), lambda i: (i, 0)),
        pl.BlockSpec((D, D), lambda i: (0, 0)),
        pl.BlockSpec((1, D), lambda i: (0, 0)),
        pl.BlockSpec((D, D), lambda i: (0, 0)),
        pl.BlockSpec((1, D), lambda i: (0, 0)),
    ],
    out_specs=pl.BlockSpec((BM, D), lambda i: (i, 0)),
    out_shape=jax.ShapeDtypeStruct((N, D), jnp.float32),
)


@jax.jit
def kernel(x, edge_index, W1, b1, W2, b2, eps):
    eflat = edge_index.astype(jnp.int32).reshape(2 * E)
    zblk = jnp.zeros((RPT_LAST, D), jnp.float32)
    p0, p1 = _sc_agg(x, eflat, zblk)
    scale = (1.0 + eps).reshape(1, 1)
    return _mlp(scale, x, p0, p1, W1, b1.reshape(1, D),
                W2, b2.reshape(1, D))


# speedup vs baseline: 1.0400x; 1.0400x over previous
import functools

import jax
import jax.numpy as jnp
from jax import lax
from jax.experimental import pallas as pl
from jax.experimental.pallas import tpu as pltpu
from jax.experimental.pallas import tpu_sc as plsc

N = 10000
E = 320000
D = 128
NC = 2
NS = 16
NW = NC * NS
EPW = E // NW
CH = 40
NCH = EPW // CH
RPT = 624
RPT_LAST = N - 15 * RPT


NB = 9
LI = 7
LG = 5


def _sc_agg_body(x_hbm, e_hbm, z_hbm, out0_hbm, out1_hbm, *scr):
    src_c = scr[0:NB]
    dst_c = scr[NB:2 * NB]
    rows = scr[2 * NB:3 * NB]
    acc = scr[3 * NB]
    isems = scr[3 * NB + 1:3 * NB + 1 + NB]
    gsems = scr[3 * NB + 1 + NB:3 * NB + 1 + 2 * NB]
    ssems = scr[3 * NB + 1 + 2 * NB:3 * NB + 1 + 3 * NB]
    c = lax.axis_index("c")
    s = lax.axis_index("s")
    wid = s * NC + c
    ebase = wid * EPW

    zsem = scr[3 * NB + 1 + 3 * NB]

    @pl.when(s < 15)
    def _():
        pltpu.async_copy(z_hbm.at[pl.ds(0, RPT)],
                         acc.at[pl.ds(s * RPT, RPT)], zsem)

    @pl.when(s == 15)
    def _():
        pltpu.async_copy(z_hbm, acc.at[pl.ds(15 * RPT, RPT_LAST)], zsem)

    def fire_idx(b, k):
        pltpu.async_copy(e_hbm.at[pl.ds(ebase + k * CH, CH)],
                         src_c[b], isems[b])
        pltpu.async_copy(e_hbm.at[pl.ds(E + ebase + k * CH, CH)],
                         dst_c[b], isems[b])

    def wait_idx(b, k):
        pltpu.make_async_copy(e_hbm.at[pl.ds(ebase + k * CH, CH)],
                              src_c[b], isems[b]).wait()
        pltpu.make_async_copy(e_hbm.at[pl.ds(E + ebase + k * CH, CH)],
                              dst_c[b], isems[b]).wait()

    def fire_gather(b):
        pltpu.async_copy(x_hbm.at[src_c[b]], rows[b], gsems[b])

    def wait_gather(b):
        pltpu.make_async_copy(x_hbm.at[src_c[b]], rows[b], gsems[b]).wait()

    def fire_scatter(b):
        pltpu.async_copy(rows[b], acc.at[dst_c[b]], ssems[b], add=True)

    def wait_scatter(b):
        pltpu.make_async_copy(rows[b], acc.at[dst_c[b]], ssems[b]).wait()

    for k in range(LI):
        fire_idx(k % NB, k)
    for k in range(LG):
        wait_idx(k % NB, k)
        fire_gather(k % NB)

    @pl.when(s < 15)
    def _():
        pltpu.make_async_copy(z_hbm.at[pl.ds(0, RPT)],
                              acc.at[pl.ds(s * RPT, RPT)], zsem).wait()

    @pl.when(s == 15)
    def _():
        pltpu.make_async_copy(z_hbm, acc.at[pl.ds(15 * RPT, RPT_LAST)],
                              zsem).wait()

    plsc.subcore_barrier()

    @pl.loop(0, ((NCH + NB - 1) // NB) * NB, step=NB)
    def _grp(i):
        for b in range(NB):
            j = i + b

            @pl.when(j < NCH)
            def _():
                @pl.when(j >= NB - LI)
                def _():
                    wait_scatter((b + LI) % NB)

                @pl.when(j + LI < NCH)
                def _():
                    fire_idx((b + LI) % NB, j + LI)

                @pl.when(j + LG < NCH)
                def _():
                    wait_idx((b + LG) % NB, j + LG)
                    fire_gather((b + LG) % NB)

                wait_gather(b)
                fire_scatter(b)

    for jj in range(NCH - (NB - LI), NCH):
        wait_scatter(jj % NB)

    plsc.subcore_barrier()

    @pl.when(jnp.logical_and(s < 15, c == 0))
    def _():
        pltpu.sync_copy(acc.at[pl.ds(s * RPT, RPT)],
                        out0_hbm.at[pl.ds(s * RPT, RPT)])

    @pl.when(jnp.logical_and(s == 15, c == 0))
    def _():
        pltpu.sync_copy(acc.at[pl.ds(15 * RPT, RPT_LAST)],
                        out0_hbm.at[pl.ds(15 * RPT, RPT_LAST)])

    @pl.when(jnp.logical_and(s < 15, c == 1))
    def _():
        pltpu.sync_copy(acc.at[pl.ds(s * RPT, RPT)],
                        out1_hbm.at[pl.ds(s * RPT, RPT)])

    @pl.when(jnp.logical_and(s == 15, c == 1))
    def _():
        pltpu.sync_copy(acc.at[pl.ds(15 * RPT, RPT_LAST)],
                        out1_hbm.at[pl.ds(15 * RPT, RPT_LAST)])


_sc_agg = functools.partial(
    pl.kernel,
    out_type=(jax.ShapeDtypeStruct((N, D), jnp.float32),
              jax.ShapeDtypeStruct((N, D), jnp.float32)),
    mesh=plsc.VectorSubcoreMesh(core_axis_name="c", subcore_axis_name="s",
                                num_cores=NC, num_subcores=NS),
    scratch_types=(
        [pltpu.VMEM((CH,), jnp.int32)] * (2 * NB)
        + [pltpu.VMEM((CH, D), jnp.float32)] * NB
        + [pltpu.VMEM_SHARED((N, D), jnp.float32)]
        + [pltpu.SemaphoreType.DMA] * (3 * NB + 1)
    ),
)(_sc_agg_body)


def _mlp_body(scale_ref, x_ref, p0_ref, p1_ref, w1_ref, b1_ref, w2_ref,
              b2_ref, o_ref):
    z = scale_ref[0, 0] * x_ref[...] + p0_ref[...] + p1_ref[...]
    h = lax.dot_general(z, w1_ref[...], (((1,), (1,)), ((), ())),
                        preferred_element_type=jnp.float32)
    h = jnp.maximum(h + b1_ref[...], 0.0)
    o = lax.dot_general(h, w2_ref[...], (((1,), (1,)), ((), ())),
                        preferred_element_type=jnp.float32)
    o_ref[...] = o + b2_ref[...]


BM = 2000

_mlp = pl.pallas_call(
    _mlp_body,
    grid=(N // BM,),
    compiler_params=pltpu.CompilerParams(
        dimension_semantics=("parallel",)),
    in_specs=[
        pl.BlockSpec(memory_space=pltpu.SMEM),
        pl.BlockSpec((BM, D), lambda i: (i, 0)),
        pl.BlockSpec((BM, D), lambda i: (i, 0)),
        pl.BlockSpec((BM, D), lambda i: (i, 0)),
        pl.BlockSpec((D, D), lambda i: (0, 0)),
        pl.BlockSpec((1, D), lambda i: (0, 0)),
        pl.BlockSpec((D, D), lambda i: (0, 0)),
        pl.BlockSpec((1, D), lambda i: (0, 0)),
    ],
    out_specs=pl.BlockSpec((BM, D), lambda i: (i, 0)),
    out_shape=jax.ShapeDtypeStruct((N, D), jnp.float32),
)


@jax.jit
def kernel(x, edge_index, W1, b1, W2, b2, eps):
    eflat = edge_index.astype(jnp.int32).reshape(2 * E)
    zblk = jnp.zeros((RPT_LAST, D), jnp.float32)
    p0, p1 = _sc_agg(x, eflat, zblk)
    scale = (1.0 + eps).reshape(1, 1)
    return _mlp(scale, x, p0, p1, W1, b1.reshape(1, D),
                W2, b2.reshape(1, D))
